# baseline (speedup 1.0000x reference)
import jax
import jax.numpy as jnp
from jax.experimental import pallas as pl


_DB = 256


def _body(in_ref, pe_ref, out_ref):
    out_ref[...] = in_ref[...] + jnp.transpose(pe_ref[...], (1, 0))[None]


def kernel(input, pe_weight):
    B, D, T = input.shape
    db = _DB
    return pl.pallas_call(
        _body,
        grid=(D // db, B),
        in_specs=[
            pl.BlockSpec((1, db, T), lambda d, b: (b, d, 0)),
            pl.BlockSpec((T, db), lambda d, b: (0, d)),
        ],
        out_specs=pl.BlockSpec((1, db, T), lambda d, b: (b, d, 0)),
        out_shape=jax.ShapeDtypeStruct(input.shape, input.dtype),
    )(input, pe_weight)


# speedup vs baseline: 1.0489x; 1.0489x over previous
import jax
import jax.numpy as jnp
from jax.experimental import pallas as pl


_DB = 512


def _body(in_ref, pe_ref, out_ref):
    out_ref[...] = in_ref[...] + jnp.transpose(pe_ref[...], (1, 0))[None]


def kernel(input, pe_weight):
    B, D, T = input.shape
    db = _DB
    return pl.pallas_call(
        _body,
        grid=(D // db, B),
        in_specs=[
            pl.BlockSpec((1, db, T), lambda d, b: (b, d, 0)),
            pl.BlockSpec((T, db), lambda d, b: (0, d)),
        ],
        out_specs=pl.BlockSpec((1, db, T), lambda d, b: (b, d, 0)),
        out_shape=jax.ShapeDtypeStruct(input.shape, input.dtype),
    )(input, pe_weight)
